# Initial kernel scaffold; baseline (speedup 1.0000x reference)
import jax
import jax.numpy as jnp
from jax.experimental import pallas as pl


def kernel(x, edge_index, W1_l, W1_r, b1, W2_l, W2_r, b2):
    raise NotImplementedError("write your pallas kernel here")


# speedup vs baseline: 6.8246x; 6.8246x over previous
import functools

import jax
import jax.numpy as jnp
from jax import lax
from jax.experimental import pallas as pl
from jax.experimental.pallas import tpu as pltpu
from jax.experimental.pallas import tpu_sc as plsc

FEAT = 128
CNT_W = 16
NC, NS = 2, 16
NW = NC * NS
ZROWS = 64


def _chunk_size(epw: int) -> int:
  for c in range(128, 7, -8):
    if epw % c == 0:
      return c
  return 8


def _pad_rows(n: int) -> int:
  q = NW * ZROWS
  return ((n + q - 1) // q) * q


def _fill(buf, nrows, width, value):
  v = jnp.full((16,), value, jnp.float32)
  per = width // 16

  def st(k, carry):
    buf[k // per, pl.ds((k % per) * 16, 16)] = v
    return carry

  lax.fori_loop(0, nrows * per, st, 0)


@functools.lru_cache(maxsize=None)
def _sc_aggregate(e_total: int, n_pad: int, feat: int = FEAT):
  epw = e_total // NW
  c_sz = _chunk_size(epw)
  it = epw // c_sz
  rpt = n_pad // NS

  mesh = plsc.VectorSubcoreMesh(
      core_axis_name="c", subcore_axis_name="s", num_cores=NC, num_subcores=NS
  )

  def body(x_hbm, src_hbm, dst_hbm, part_hbm,
           acc_sh, src_idx, dst_idx, rows, zbuf, sem):
    cid = lax.axis_index("c")
    sid = lax.axis_index("s")
    wid = sid * NC + cid
    row0 = sid * rpt

    _fill(zbuf, ZROWS, feat, 0.0)
    for k in range(rpt // ZROWS):
      pltpu.sync_copy(zbuf, acc_sh.at[pl.ds(row0 + k * ZROWS, ZROWS)])

    pltpu.sync_copy(src_hbm.at[pl.ds(wid * epw, epw)], src_idx)
    pltpu.sync_copy(dst_hbm.at[wid], dst_idx)

    plsc.subcore_barrier()

    def step(i, carry):
      pltpu.async_copy(
          x_hbm.at[src_idx.at[pl.ds(i * c_sz, c_sz)]], rows, sem
      ).wait()
      pltpu.sync_copy(rows, acc_sh.at[dst_idx.at[i]], add=True)
      return carry

    lax.fori_loop(0, it, step, 0)

    plsc.subcore_barrier()

    pltpu.sync_copy(acc_sh.at[pl.ds(row0, rpt)],
                    part_hbm.at[cid, pl.ds(row0, rpt)])

  return pl.kernel(
      body,
      out_type=[jax.ShapeDtypeStruct((NC, n_pad, feat), jnp.float32)],
      mesh=mesh,
      scratch_types=[
          pltpu.VMEM_SHARED((n_pad, feat), jnp.float32),
          pltpu.VMEM((epw,), jnp.int32),
          pltpu.VMEM((it, c_sz), jnp.int32),
          pltpu.VMEM((c_sz, feat), jnp.float32),
          pltpu.VMEM((ZROWS, feat), jnp.float32),
          pltpu.SemaphoreType.DMA,
      ],
  )


@functools.lru_cache(maxsize=None)
def _sc_counts(e_total: int, n_pad: int):
  epw = e_total // NW
  c_sz = _chunk_size(epw)
  it = epw // c_sz
  rpt = n_pad // NS

  mesh = plsc.VectorSubcoreMesh(
      core_axis_name="c", subcore_axis_name="s", num_cores=NC, num_subcores=NS
  )

  def body(dst_hbm, ones_hbm, cnt_hbm, cnt_sh, dst_idx, ones_v, zbuf):
    cid = lax.axis_index("c")
    sid = lax.axis_index("s")
    wid = sid * NC + cid
    row0 = sid * rpt

    _fill(zbuf, ZROWS, FEAT, 0.0)
    for k in range(rpt // ZROWS):
      pltpu.sync_copy(zbuf, cnt_sh.at[pl.ds(row0 + k * ZROWS, ZROWS)])
    pltpu.sync_copy(ones_hbm, ones_v)
    pltpu.sync_copy(dst_hbm.at[wid], dst_idx)

    plsc.subcore_barrier()

    def step(i, carry):
      pltpu.sync_copy(ones_v, cnt_sh.at[dst_idx.at[i]], add=True)
      return carry

    lax.fori_loop(0, it, step, 0)

    plsc.subcore_barrier()

    pltpu.sync_copy(cnt_sh.at[pl.ds(row0, rpt)],
                    cnt_hbm.at[cid, pl.ds(row0, rpt)])

  return pl.kernel(
      body,
      out_type=[jax.ShapeDtypeStruct((NC, n_pad, FEAT), jnp.float32)],
      mesh=mesh,
      scratch_types=[
          pltpu.VMEM_SHARED((n_pad, FEAT), jnp.float32),
          pltpu.VMEM((it, c_sz), jnp.int32),
          pltpu.VMEM((c_sz, FEAT), jnp.float32),
          pltpu.VMEM((ZROWS, FEAT), jnp.float32),
      ],
  )


def _dense_layer(part, cnt, xin, w_l_t, w_r_t, b, activation):
  n_pad = xin.shape[0]
  rblk = 1024
  grid = (n_pad // rblk,)

  def body(p_ref, c_ref, x_ref, wl_ref, wr_ref, b_ref, o_ref):
    ssum = p_ref[0] + p_ref[1]
    cvec = c_ref[0, :, 0:1] + c_ref[1, :, 0:1]
    mean = ssum / jnp.maximum(cvec, 1.0)
    y = (
        jnp.dot(mean, wl_ref[...], preferred_element_type=jnp.float32,
                precision=lax.Precision.HIGHEST)
        + jnp.dot(x_ref[...], wr_ref[...], preferred_element_type=jnp.float32,
                  precision=lax.Precision.HIGHEST)
        + b_ref[...]
    )
    if activation == "relu":
      o_ref[...] = jnp.maximum(y, 0.0)
    else:
      m = jnp.max(y, axis=1, keepdims=True)
      e = jnp.exp(y - m)
      o_ref[...] = e / jnp.sum(e, axis=1, keepdims=True)

  return pl.pallas_call(
      body,
      grid=grid,
      in_specs=[
          pl.BlockSpec((NC, rblk, FEAT), lambda i: (0, i, 0)),
          pl.BlockSpec((NC, rblk, FEAT), lambda i: (0, i, 0)),
          pl.BlockSpec((rblk, FEAT), lambda i: (i, 0)),
          pl.BlockSpec((FEAT, FEAT), lambda i: (0, 0)),
          pl.BlockSpec((FEAT, FEAT), lambda i: (0, 0)),
          pl.BlockSpec((1, FEAT), lambda i: (0, 0)),
      ],
      out_specs=pl.BlockSpec((rblk, FEAT), lambda i: (i, 0)),
      out_shape=jax.ShapeDtypeStruct((n_pad, FEAT), jnp.float32),
  )(part, cnt, xin, w_l_t, w_r_t, b)


def kernel(x, edge_index, W1_l, W1_r, b1, W2_l, W2_r, b2):
  x = x.astype(jnp.float32)
  src = edge_index[0].astype(jnp.int32)
  dst = edge_index[1].astype(jnp.int32)
  e_total = src.shape[0]
  n = x.shape[0]
  n_pad = _pad_rows(n)

  epw = e_total // NW
  c_sz = _chunk_size(epw)
  dst3 = dst.reshape(NW, epw // c_sz, c_sz)

  xp = jnp.zeros((n_pad, FEAT), jnp.float32).at[:n].set(x)
  ones_in = jnp.ones((c_sz, FEAT), jnp.float32)

  (cnt,) = _sc_counts(e_total, n_pad)(dst3, ones_in)
  (part1,) = _sc_aggregate(e_total, n_pad)(xp, src, dst3)
  h = _dense_layer(part1, cnt, xp, W1_l.T, W1_r.T, b1.reshape(1, FEAT), "relu")
  (part2,) = _sc_aggregate(e_total, n_pad)(h, src, dst3)
  out = _dense_layer(part2, cnt, h, W2_l.T, W2_r.T, b2.reshape(1, FEAT),
                     "softmax")
  return out[:n]


# speedup vs baseline: 8.3004x; 1.2162x over previous
import functools

import jax
import jax.numpy as jnp
from jax import lax
from jax.experimental import pallas as pl
from jax.experimental.pallas import tpu as pltpu
from jax.experimental.pallas import tpu_sc as plsc

FEAT = 128
CNT_W = 16
NC, NS = 2, 16
NW = NC * NS
ZROWS = 64


def _chunk_size(epw: int) -> int:
  for c in range(128, 7, -8):
    if epw % c == 0:
      return c
  return 8


def _pad_rows(n: int) -> int:
  q = NW * ZROWS
  return ((n + q - 1) // q) * q


@functools.lru_cache(maxsize=None)
def _sc_aggregate(e_total: int, n_pad: int, feat: int = FEAT):
  epw = e_total // NW
  c_sz = _chunk_size(epw)
  it = epw // c_sz
  rpt = n_pad // NS

  mesh = plsc.VectorSubcoreMesh(
      core_axis_name="c", subcore_axis_name="s", num_cores=NC, num_subcores=NS
  )

  def body(x_hbm, src_hbm, dst_hbm, zer_hbm, part_hbm,
           acc_sh, src_idx, dst_idx, rows0, rows1, sg0, sg1):
    ss0, ss1 = sg0, sg1
    cid = lax.axis_index("c")
    sid = lax.axis_index("s")
    wid = sid * NC + cid
    row0 = sid * rpt

    def gather(i, buf, sem):
      return pltpu.async_copy(
          x_hbm.at[src_idx.at[pl.ds(i * c_sz, c_sz)]], buf, sem)

    def gather_wait(i, buf, sem):
      pltpu.make_async_copy(
          x_hbm.at[src_idx.at[pl.ds(i * c_sz, c_sz)]], buf, sem).wait()

    def scat(i, buf, sem):
      return pltpu.async_copy(buf, acc_sh.at[dst_idx.at[i]], sem, add=True)

    def scat_wait(i, buf, sem):
      pltpu.make_async_copy(buf, acc_sh.at[dst_idx.at[i]], sem).wait()

    pltpu.sync_copy(zer_hbm, acc_sh.at[pl.ds(row0, rpt)])

    pltpu.sync_copy(src_hbm.at[pl.ds(wid * epw, epw)], src_idx)
    pltpu.sync_copy(dst_hbm.at[wid], dst_idx)

    plsc.subcore_barrier()

    pairs = it // 2
    gather(0, rows0, sg0)
    if it > 1:
      gather(1, rows1, sg1)

    def pair(j, carry):
      i0 = j * 2
      i1 = i0 + 1
      gather_wait(i0, rows0, sg0)
      scat(i0, rows0, ss0)
      gather_wait(i1, rows1, sg1)
      scat(i1, rows1, ss1)
      scat_wait(i0, rows0, ss0)

      @pl.when(i0 + 2 < it)
      def _():
        gather(i0 + 2, rows0, sg0)

      scat_wait(i1, rows1, ss1)

      @pl.when(i1 + 2 < it)
      def _():
        gather(i1 + 2, rows1, sg1)

      return carry

    lax.fori_loop(0, pairs, pair, 0)

    if it % 2 == 1:
      i_last = it - 1
      gather_wait(i_last, rows0, sg0)
      scat(i_last, rows0, ss0)
      scat_wait(i_last, rows0, ss0)

    plsc.subcore_barrier()

    pltpu.sync_copy(acc_sh.at[pl.ds(row0, rpt)],
                    part_hbm.at[cid, pl.ds(row0, rpt)])

  return pl.kernel(
      body,
      out_type=[jax.ShapeDtypeStruct((NC, n_pad, feat), jnp.float32)],
      mesh=mesh,
      scratch_types=[
          pltpu.VMEM_SHARED((n_pad, feat), jnp.float32),
          pltpu.VMEM((epw,), jnp.int32),
          pltpu.VMEM((it, c_sz), jnp.int32),
          pltpu.VMEM((c_sz, feat), jnp.float32),
          pltpu.VMEM((c_sz, feat), jnp.float32),
          pltpu.SemaphoreType.DMA,
          pltpu.SemaphoreType.DMA,
      ],
  )


@functools.lru_cache(maxsize=None)
def _sc_counts(e_total: int, n_pad: int):
  epw = e_total // NW
  c_sz = _chunk_size(epw)
  it = epw // c_sz
  rpt = n_pad // NS

  mesh = plsc.VectorSubcoreMesh(
      core_axis_name="c", subcore_axis_name="s", num_cores=NC, num_subcores=NS
  )

  def body(dst_hbm, ones_hbm, zer_hbm, cnt_hbm, cnt_sh, dst_idx, ones_v, sem):
    cid = lax.axis_index("c")
    sid = lax.axis_index("s")
    wid = sid * NC + cid
    row0 = sid * rpt

    pltpu.sync_copy(zer_hbm, cnt_sh.at[pl.ds(row0, rpt)])
    pltpu.sync_copy(ones_hbm, ones_v)
    pltpu.sync_copy(dst_hbm.at[wid], dst_idx)

    plsc.subcore_barrier()

    def fire(i, carry):
      pltpu.async_copy(ones_v, cnt_sh.at[dst_idx.at[i]], sem, add=True)
      return carry

    def drain(i, carry):
      pltpu.make_async_copy(ones_v, cnt_sh.at[dst_idx.at[i]], sem).wait()
      return carry

    lax.fori_loop(0, it, fire, 0)
    lax.fori_loop(0, it, drain, 0)

    plsc.subcore_barrier()

    pltpu.sync_copy(cnt_sh.at[pl.ds(row0, rpt)],
                    cnt_hbm.at[cid, pl.ds(row0, rpt)])

  return pl.kernel(
      body,
      out_type=[jax.ShapeDtypeStruct((NC, n_pad, FEAT), jnp.float32)],
      mesh=mesh,
      scratch_types=[
          pltpu.VMEM_SHARED((n_pad, FEAT), jnp.float32),
          pltpu.VMEM((it, c_sz), jnp.int32),
          pltpu.VMEM((c_sz, FEAT), jnp.float32),
          pltpu.SemaphoreType.DMA,
      ],
  )


def _dense_layer(part, cnt, xin, w_l_t, w_r_t, b, activation):
  n = xin.shape[0]
  rblk = 1000
  grid = (n // rblk,)

  def body(p_ref, c_ref, x_ref, wl_ref, wr_ref, b_ref, o_ref):
    ssum = p_ref[0] + p_ref[1]
    cvec = c_ref[0, :, 0:1] + c_ref[1, :, 0:1]
    mean = ssum / jnp.maximum(cvec, 1.0)
    y = (
        jnp.dot(mean, wl_ref[...], preferred_element_type=jnp.float32,
                precision=lax.Precision.HIGHEST)
        + jnp.dot(x_ref[...], wr_ref[...], preferred_element_type=jnp.float32,
                  precision=lax.Precision.HIGHEST)
        + b_ref[...]
    )
    if activation == "relu":
      o_ref[...] = jnp.maximum(y, 0.0)
    else:
      m = jnp.max(y, axis=1, keepdims=True)
      e = jnp.exp(y - m)
      o_ref[...] = e / jnp.sum(e, axis=1, keepdims=True)

  return pl.pallas_call(
      body,
      grid=grid,
      in_specs=[
          pl.BlockSpec((NC, rblk, FEAT), lambda i: (0, i, 0)),
          pl.BlockSpec((NC, rblk, FEAT), lambda i: (0, i, 0)),
          pl.BlockSpec((rblk, FEAT), lambda i: (i, 0)),
          pl.BlockSpec((FEAT, FEAT), lambda i: (0, 0)),
          pl.BlockSpec((FEAT, FEAT), lambda i: (0, 0)),
          pl.BlockSpec((1, FEAT), lambda i: (0, 0)),
      ],
      out_specs=pl.BlockSpec((rblk, FEAT), lambda i: (i, 0)),
      out_shape=jax.ShapeDtypeStruct((n, FEAT), jnp.float32),
  )(part, cnt, xin, w_l_t, w_r_t, b)


def kernel(x, edge_index, W1_l, W1_r, b1, W2_l, W2_r, b2):
  x = x.astype(jnp.float32)
  src = edge_index[0].astype(jnp.int32)
  dst = edge_index[1].astype(jnp.int32)
  e_total = src.shape[0]
  n = x.shape[0]
  n_pad = _pad_rows(n)

  epw = e_total // NW
  c_sz = _chunk_size(epw)
  dst3 = dst.reshape(NW, epw // c_sz, c_sz)

  ones_in = jnp.ones((c_sz, FEAT), jnp.float32)
  zer_in = jnp.zeros((n_pad // NS, FEAT), jnp.float32)

  (cnt,) = _sc_counts(e_total, n_pad)(dst3, ones_in, zer_in)
  (part1,) = _sc_aggregate(e_total, n_pad)(x, src, dst3, zer_in)
  h = _dense_layer(part1, cnt, x, W1_l.T, W1_r.T, b1.reshape(1, FEAT), "relu")
  (part2,) = _sc_aggregate(e_total, n_pad)(h, src, dst3, zer_in)
  return _dense_layer(part2, cnt, h, W2_l.T, W2_r.T, b2.reshape(1, FEAT),
                      "softmax")
